# baseline (speedup 1.0000x reference)
import jax
import jax.numpy as jnp
from jax.experimental import pallas as pl
from jax.experimental.pallas import tpu as pltpu

_B_BLK = 2048


def _cdist_block(feat_ref, centers_ref, dt_ref, lt_ref, pred_ref,
                 cs_ref, c2_ref):
    @pl.when(pl.program_id(0) == 0)
    def _init():
        c = centers_ref[...]
        cs_ref[...] = c * -2.0
        c2_ref[...] = jnp.sum(c * c, axis=1, keepdims=True)

    f = feat_ref[...]
    ones = jnp.ones((1, f.shape[1]), jnp.float32)
    f2 = jax.lax.dot_general(
        ones, f * f, (((1,), (1,)), ((), ())), preferred_element_type=jnp.float32
    )
    dotm2 = jax.lax.dot_general(
        cs_ref[...], f, (((1,), (1,)), ((), ())),
        preferred_element_type=jnp.float32,
    )
    d2 = dotm2 + (c2_ref[...] + f2)
    dists = jnp.sqrt(jnp.maximum(d2, 1e-12))
    dt_ref[...] = dists
    lt_ref[...] = -dists
    pred_ref[...] = jnp.argmin(d2, axis=0).astype(jnp.int32)[None, :]


def kernel(feat, centers):
    B, D = feat.shape
    K = centers.shape[0]
    grid = (B // _B_BLK,)
    dists_t, logits_t, pred = pl.pallas_call(
        _cdist_block,
        grid=grid,
        in_specs=[
            pl.BlockSpec((_B_BLK, D), lambda i: (i, 0)),
            pl.BlockSpec((K, D), lambda i: (0, 0)),
        ],
        out_specs=[
            pl.BlockSpec((K, _B_BLK), lambda i: (0, i)),
            pl.BlockSpec((K, _B_BLK), lambda i: (0, i)),
            pl.BlockSpec((1, _B_BLK), lambda i: (0, i)),
        ],
        out_shape=[
            jax.ShapeDtypeStruct((K, B), jnp.float32),
            jax.ShapeDtypeStruct((K, B), jnp.float32),
            jax.ShapeDtypeStruct((1, B), jnp.int32),
        ],
        scratch_shapes=[
            pltpu.VMEM((K, D), jnp.float32),
            pltpu.VMEM((K, 1), jnp.float32),
        ],
    )(feat, centers)
    return (dists_t.T, logits_t.T, pred[0])


# speedup vs baseline: 1.0515x; 1.0515x over previous
import jax
import jax.numpy as jnp
from jax.experimental import pallas as pl
from jax.experimental.pallas import tpu as pltpu

_B_BLK = 2048


def _cdist_block(feat_ref, centers_ref, dt_ref, lt_ref, pred_ref,
                 cs_ref, c2_ref):
    @pl.when(pl.program_id(0) == 0)
    def _init():
        c = centers_ref[...]
        cs_ref[...] = c * -2.0
        c2 = jnp.sum(c * c, axis=1, keepdims=True)
        c2_ref[...] = jnp.concatenate(
            [jnp.ones_like(c2), c2], axis=1)

    f = feat_ref[...]
    f2 = jnp.sum(f * f, axis=1, keepdims=True)
    f2e = jnp.concatenate([f2, jnp.ones_like(f2)], axis=1)
    fc2 = jax.lax.dot_general(
        c2_ref[...], f2e, (((1,), (1,)), ((), ())),
        preferred_element_type=jnp.float32,
    )
    dotm2 = jax.lax.dot_general(
        cs_ref[...], f, (((1,), (1,)), ((), ())),
        preferred_element_type=jnp.float32,
    )
    d2 = dotm2 + fc2
    d2c = jnp.maximum(d2, 1e-12)
    dists = d2c * jax.lax.rsqrt(d2c)
    dt_ref[...] = dists
    lt_ref[...] = -dists
    pred_ref[...] = jnp.argmin(d2, axis=0).astype(jnp.int32)[None, :]


def kernel(feat, centers):
    B, D = feat.shape
    K = centers.shape[0]
    grid = (B // _B_BLK,)
    dists_t, logits_t, pred = pl.pallas_call(
        _cdist_block,
        grid=grid,
        in_specs=[
            pl.BlockSpec((_B_BLK, D), lambda i: (i, 0)),
            pl.BlockSpec((K, D), lambda i: (0, 0)),
        ],
        out_specs=[
            pl.BlockSpec((K, _B_BLK), lambda i: (0, i)),
            pl.BlockSpec((K, _B_BLK), lambda i: (0, i)),
            pl.BlockSpec((1, _B_BLK), lambda i: (0, i)),
        ],
        out_shape=[
            jax.ShapeDtypeStruct((K, B), jnp.float32),
            jax.ShapeDtypeStruct((K, B), jnp.float32),
            jax.ShapeDtypeStruct((1, B), jnp.int32),
        ],
        scratch_shapes=[
            pltpu.VMEM((K, D), jnp.float32),
            pltpu.VMEM((K, 2), jnp.float32),
        ],
    )(feat, centers)
    return (dists_t.T, logits_t.T, pred[0])


# speedup vs baseline: 1.0695x; 1.0171x over previous
import jax
import jax.numpy as jnp
from jax.experimental import pallas as pl
from jax.experimental.pallas import tpu as pltpu

_B_BLK = 2048


def _cdist_block(feat_ref, centers_ref, dt_ref, lt_ref, pred_ref,
                 cs_ref, c2_ref):
    @pl.when(pl.program_id(0) == 0)
    def _init():
        c = centers_ref[...]
        cs_ref[...] = c * -2.0
        c2_ref[...] = jnp.sum(c * c, axis=1, keepdims=True)

    f = feat_ref[...]
    ones = jnp.ones((1, f.shape[1]), jnp.float32)
    f2 = jax.lax.dot_general(
        ones, f * f, (((1,), (1,)), ((), ())), preferred_element_type=jnp.float32
    )
    dotm2 = jax.lax.dot_general(
        cs_ref[...], f, (((1,), (1,)), ((), ())),
        preferred_element_type=jnp.float32,
    )
    d2 = dotm2 + (c2_ref[...] + f2)
    d2c = jnp.maximum(d2, 1e-12)
    dists = d2c * jax.lax.rsqrt(d2c)
    dt_ref[...] = dists
    lt_ref[...] = -dists
    pred_ref[...] = jnp.argmin(d2, axis=0).astype(jnp.int32)[None, :]


def kernel(feat, centers):
    B, D = feat.shape
    K = centers.shape[0]
    grid = (B // _B_BLK,)
    dists_t, logits_t, pred = pl.pallas_call(
        _cdist_block,
        grid=grid,
        in_specs=[
            pl.BlockSpec((_B_BLK, D), lambda i: (i, 0)),
            pl.BlockSpec((K, D), lambda i: (0, 0)),
        ],
        out_specs=[
            pl.BlockSpec((K, _B_BLK), lambda i: (0, i)),
            pl.BlockSpec((K, _B_BLK), lambda i: (0, i)),
            pl.BlockSpec((1, _B_BLK), lambda i: (0, i)),
        ],
        out_shape=[
            jax.ShapeDtypeStruct((K, B), jnp.float32),
            jax.ShapeDtypeStruct((K, B), jnp.float32),
            jax.ShapeDtypeStruct((1, B), jnp.int32),
        ],
        scratch_shapes=[
            pltpu.VMEM((K, D), jnp.float32),
            pltpu.VMEM((K, 1), jnp.float32),
        ],
    )(feat, centers)
    return (dists_t.T, logits_t.T, pred[0])
